# Initial kernel scaffold; baseline (speedup 1.0000x reference)
import jax
import jax.numpy as jnp
from jax.experimental import pallas as pl


def kernel(x, edge_index, edge_weight, hidden_state, Wz0, Wz1, bz, Wr0, Wr1, br, Wh0, Wh1, bh):
    raise NotImplementedError("write your pallas kernel here")


# speedup vs baseline: 6.2878x; 6.2878x over previous
import functools

import jax
import jax.numpy as jnp
from jax import lax
from jax.experimental import pallas as pl
from jax.experimental.pallas import tpu as pltpu
from jax.experimental.pallas import tpu_sc as plsc

N = 10000
E = 320000
C = 128
NPAD = 10240
NC = 2
NS = 16
NW = NC * NS
CH = 80
NROWS = E // CH
L = 16

_mesh = lambda: plsc.VectorSubcoreMesh(core_axis_name="c", subcore_axis_name="s")


def _deg_kernel(row_h, ew_h, out_h, rowc_v, ewc_v, zb_v, dacc_s, sem):
    c = lax.axis_index("c")
    s = lax.axis_index("s")
    w = c * NS + s
    nchunks = NROWS // NW
    nsl = NPAD // NS

    def zero_body(i, _):
        zb_v[pl.ds(i * L, L)] = jnp.zeros((L,), jnp.float32)
        return 0
    lax.fori_loop(0, nsl // L, zero_body, 0)
    pltpu.sync_copy(zb_v, dacc_s.at[pl.ds(s * nsl, nsl)])
    plsc.subcore_barrier()

    def body(j, _):
        k = w * nchunks + j
        pltpu.sync_copy(row_h.at[k], rowc_v)
        pltpu.sync_copy(ew_h.at[k], ewc_v)
        pltpu.sync_copy(ewc_v, dacc_s.at[rowc_v], add=True)
        return 0
    lax.fori_loop(0, nchunks, body, 0)

    plsc.subcore_barrier()
    pltpu.sync_copy(dacc_s.at[pl.ds(s * nsl, nsl)],
                    out_h.at[pl.ds(c * NPAD + s * nsl, nsl)])


def _deg_partials(row2, ew2):
    f = pl.kernel(
        _deg_kernel,
        out_type=jax.ShapeDtypeStruct((NC * NPAD,), jnp.float32),
        mesh=_mesh(),
        compiler_params=pltpu.CompilerParams(needs_layout_passes=False),
        scratch_types=[
            pltpu.VMEM((CH,), jnp.int32),
            pltpu.VMEM((CH,), jnp.float32),
            pltpu.VMEM((NPAD // NS,), jnp.float32),
            pltpu.VMEM_SHARED((NPAD,), jnp.float32),
            pltpu.SemaphoreType.DMA,
        ],
    )
    return f(row2, ew2)


def _dis_body(dp_ref, out_ref):
    deg = jnp.sum(dp_ref[...], axis=0)
    out_ref[...] = jnp.where(deg > 0, lax.rsqrt(deg), 0.0)


def _dis_tc(deg_parts):
    dp3 = deg_parts.reshape(NC, 8, NPAD // 8)
    out = pl.pallas_call(
        _dis_body,
        out_shape=jax.ShapeDtypeStruct((8, NPAD // 8), jnp.float32),
    )(dp3)
    return out.reshape(NPAD)


def _agg_xh_kernel(row_h, col_h, ew_h, dis_h, xh_h, zeros_h,
                   a_out_h, norm_out_h,
                   dis_v, rowc_v, colc_v, ewc_v, normc_v, rows_v, acc_s, sem):
    c = lax.axis_index("c")
    s = lax.axis_index("s")
    nchunks = NROWS // NS

    pltpu.sync_copy(dis_h, dis_v)
    nslice = NPAD // NS
    for i in range(nslice // 128):
        pltpu.sync_copy(zeros_h, acc_s.at[pl.ds(s * nslice + i * 128, 128)])
    plsc.subcore_barrier()

    def body(j, _):
        k = s * nchunks + j
        pltpu.sync_copy(row_h.at[k], rowc_v)
        pltpu.sync_copy(col_h.at[k], colc_v)
        pltpu.sync_copy(ew_h.at[k], ewc_v)
        for g in range(CH // L):
            r16 = rowc_v[pl.ds(g * L, L)]
            c16 = colc_v[pl.ds(g * L, L)]
            w16 = ewc_v[pl.ds(g * L, L)]
            a = plsc.load_gather(dis_v, [r16])
            b = plsc.load_gather(dis_v, [c16])
            normc_v[pl.ds(g * L, L)] = -(a * w16) * b
        pltpu.sync_copy(normc_v, norm_out_h.at[k])

        off = jnp.full((L,), c * N, jnp.int32)
        for g in range(CH // L):
            rowc_v[pl.ds(g * L, L)] = rowc_v[pl.ds(g * L, L)] + off
        pltpu.async_copy(xh_h.at[rowc_v], rows_v, sem).wait()

        def scale(g, _):
            nv = normc_v[pl.ds(g * L, L)]
            for e in range(L):
                sv = jnp.full((L,), nv[e], jnp.float32)
                r = g * L + e
                for u in range(C // L):
                    rows_v[r, pl.ds(u * L, L)] = rows_v[r, pl.ds(u * L, L)] * sv
            return 0
        lax.fori_loop(0, CH // L, scale, 0)

        pltpu.sync_copy(rows_v, acc_s.at[colc_v], add=True)
        return 0
    lax.fori_loop(0, nchunks, body, 0)

    plsc.subcore_barrier()
    pltpu.sync_copy(acc_s.at[pl.ds(s * nslice, nslice)],
                    a_out_h.at[pl.ds(c * NPAD + s * nslice, nslice)])


def _agg_xh(row2, col2, ew2, dis, xh, zeros):
    f = pl.kernel(
        _agg_xh_kernel,
        out_type=(jax.ShapeDtypeStruct((2 * NPAD, C), jnp.float32),
                  jax.ShapeDtypeStruct((NROWS, CH), jnp.float32)),
        mesh=_mesh(),
        compiler_params=pltpu.CompilerParams(needs_layout_passes=False),
        scratch_types=[
            pltpu.VMEM((NPAD,), jnp.float32),
            pltpu.VMEM((CH,), jnp.int32),
            pltpu.VMEM((CH,), jnp.int32),
            pltpu.VMEM((CH,), jnp.float32),
            pltpu.VMEM((CH,), jnp.float32),
            pltpu.VMEM((CH, C), jnp.float32),
            pltpu.VMEM_SHARED((NPAD, C), jnp.float32),
            pltpu.SemaphoreType.DMA,
        ],
    )
    return f(row2, col2, ew2, dis, xh, zeros)


def _agg_rh_kernel(row_h, col_h, norm_h, rh_h, zeros_h, a_out_h,
                   rowc_v, colc_v, normc_v, rows_v, acc_s, sem):
    c = lax.axis_index("c")
    s = lax.axis_index("s")
    w = c * NS + s
    nchunks = NROWS // NW

    nslice = NPAD // NS
    for i in range(nslice // 128):
        pltpu.sync_copy(zeros_h, acc_s.at[pl.ds(s * nslice + i * 128, 128)])
    plsc.subcore_barrier()

    def body(j, _):
        k = w * nchunks + j
        pltpu.sync_copy(row_h.at[k], rowc_v)
        pltpu.sync_copy(col_h.at[k], colc_v)
        pltpu.sync_copy(norm_h.at[k], normc_v)
        pltpu.async_copy(rh_h.at[rowc_v], rows_v, sem).wait()

        def scale(g, _):
            nv = normc_v[pl.ds(g * L, L)]
            for e in range(L):
                sv = jnp.full((L,), nv[e], jnp.float32)
                r = g * L + e
                for u in range(C // L):
                    rows_v[r, pl.ds(u * L, L)] = rows_v[r, pl.ds(u * L, L)] * sv
            return 0
        lax.fori_loop(0, CH // L, scale, 0)

        pltpu.sync_copy(rows_v, acc_s.at[colc_v], add=True)
        return 0
    lax.fori_loop(0, nchunks, body, 0)

    plsc.subcore_barrier()
    pltpu.sync_copy(acc_s.at[pl.ds(s * nslice, nslice)],
                    a_out_h.at[pl.ds(c * NPAD + s * nslice, nslice)])


def _agg_rh(row2, col2, norm2, rh, zeros):
    f = pl.kernel(
        _agg_rh_kernel,
        out_type=jax.ShapeDtypeStruct((2 * NPAD, C), jnp.float32),
        mesh=_mesh(),
        compiler_params=pltpu.CompilerParams(needs_layout_passes=False),
        scratch_types=[
            pltpu.VMEM((CH,), jnp.int32),
            pltpu.VMEM((CH,), jnp.int32),
            pltpu.VMEM((CH,), jnp.float32),
            pltpu.VMEM((CH, C), jnp.float32),
            pltpu.VMEM_SHARED((NPAD, C), jnp.float32),
            pltpu.SemaphoreType.DMA,
        ],
    )
    return f(row2, col2, norm2, rh, zeros)


def _gates_body(x_r, h_r, ax_r, ah_r,
                wz0a_r, wz0b_r, wz1a_r, wz1b_r, bz_r,
                wr0a_r, wr0b_r, wr1a_r, wr1b_r, br_r,
                wh0a_r, wh1a_r, bh_r,
                z_r, rh_r, cp_r):
    dot = functools.partial(jnp.dot, preferred_element_type=jnp.float32)
    xx, hh = x_r[...], h_r[...]
    ax, ah = ax_r[...], ah_r[...]
    gz = (dot(xx, wz0a_r[...]) + dot(hh, wz0b_r[...])
          + dot(ax, wz1a_r[...]) + dot(ah, wz1b_r[...]) + bz_r[...])
    gr = (dot(xx, wr0a_r[...]) + dot(hh, wr0b_r[...])
          + dot(ax, wr1a_r[...]) + dot(ah, wr1b_r[...]) + br_r[...])
    z_r[...] = jax.nn.sigmoid(gz)
    rh_r[...] = jax.nn.sigmoid(gr) * hh
    cp_r[...] = dot(xx, wh0a_r[...]) + dot(ax, wh1a_r[...]) + bh_r[...]


def _gates_tc(x, h, ax, ah, wz0a, wz0b, wz1a, wz1b, bz2,
              wr0a, wr0b, wr1a, wr1b, br2, wh0a, wh1a, bh2):
    nb = 10
    rs = pl.BlockSpec((N // nb, C), lambda i: (i, 0))
    ws = pl.BlockSpec((C, C), lambda i: (0, 0))
    bs = pl.BlockSpec((1, C), lambda i: (0, 0))
    return pl.pallas_call(
        _gates_body,
        grid=(nb,),
        in_specs=[rs, rs, rs, rs,
                  ws, ws, ws, ws, bs,
                  ws, ws, ws, ws, bs,
                  ws, ws, bs],
        out_specs=[rs, rs, rs],
        out_shape=[jax.ShapeDtypeStruct((N, C), jnp.float32)] * 3,
    )(x, h, ax, ah, wz0a, wz0b, wz1a, wz1b, bz2,
      wr0a, wr0b, wr1a, wr1b, br2, wh0a, wh1a, bh2)


def _out_body(z_r, rh_r, cp_r, h_r, ar0_r, ar1_r, wh0b_r, wh1b_r, o_r):
    dot = functools.partial(jnp.dot, preferred_element_type=jnp.float32)
    cand = jnp.tanh(cp_r[...] + dot(rh_r[...], wh0b_r[...])
                    + dot(ar0_r[...] + ar1_r[...], wh1b_r[...]))
    z = z_r[...]
    o_r[...] = (1.0 - z) * h_r[...] + z * cand


def _out_tc(z, rh, cp, h, ar0, ar1, wh0b, wh1b):
    nb = 10
    rs = pl.BlockSpec((N // nb, C), lambda i: (i, 0))
    ws = pl.BlockSpec((C, C), lambda i: (0, 0))
    return pl.pallas_call(
        _out_body,
        grid=(nb,),
        in_specs=[rs, rs, rs, rs, rs, rs, ws, ws],
        out_specs=rs,
        out_shape=jax.ShapeDtypeStruct((N, C), jnp.float32),
    )(z, rh, cp, h, ar0, ar1, wh0b, wh1b)


def kernel(x, edge_index, edge_weight, hidden_state,
           Wz0, Wz1, bz, Wr0, Wr1, br, Wh0, Wh1, bh):
    row = edge_index[0].reshape(NROWS, CH)
    col = edge_index[1].reshape(NROWS, CH)
    ew = edge_weight.reshape(NROWS, CH)
    zeros = jnp.zeros((128, C), jnp.float32)

    deg_parts = _deg_partials(row, ew)
    dis = _dis_tc(deg_parts)

    xh = jnp.concatenate([x, hidden_state], axis=0)
    a_cat, norm2 = _agg_xh(row, col, ew, dis, xh, zeros)
    a_x, a_h = a_cat[:N], a_cat[NPAD:NPAD + N]

    z, rh, cpre = _gates_tc(
        x, hidden_state, a_x, a_h,
        Wz0[:C], Wz0[C:], Wz1[:C], Wz1[C:], bz.reshape(1, C),
        Wr0[:C], Wr0[C:], Wr1[:C], Wr1[C:], br.reshape(1, C),
        Wh0[:C], Wh1[:C], bh.reshape(1, C))

    ar = _agg_rh(row, col, norm2, rh, zeros)

    return _out_tc(z, rh, cpre, hidden_state, ar[:N], ar[NPAD:NPAD + N],
                   Wh0[C:], Wh1[C:])


# speedup vs baseline: 17.2425x; 2.7422x over previous
import functools

import jax
import jax.numpy as jnp
from jax import lax
from jax.experimental import pallas as pl
from jax.experimental.pallas import tpu as pltpu
from jax.experimental.pallas import tpu_sc as plsc

N = 10000
E = 320000
C = 128
NPAD = 10240
NC = 2
NS = 16
NW = NC * NS
CH = 80
NROWS = E // CH
L = 16
NBUF = 5

_mesh = lambda: plsc.VectorSubcoreMesh(core_axis_name="c", subcore_axis_name="s")
_params = lambda: pltpu.CompilerParams(needs_layout_passes=False)


def _rsqrt_newton(d):
    i = plsc.bitcast(d, jnp.int32)
    y = plsc.bitcast(jnp.int32(0x5F3759DF) - lax.shift_right_logical(i, 1),
                     jnp.float32)
    for _ in range(3):
        y = y * (1.5 - 0.5 * d * y * y)
    return jnp.where(d > 0, y, 0.0)


BCH = 25
EPB = BCH * CH
NBLK = E // EPB


def _agg_pipeline(tab_h, rowb_v, col2_v, normb_v, bufs, gsems, ssems, acc_s):
    def fire_gather(q, b):
        pltpu.async_copy(tab_h.at[rowb_v.at[pl.ds(q * CH, CH)]], bufs[b],
                         gsems[b])

    def wait_gather(q, b):
        pltpu.make_async_copy(tab_h.at[rowb_v.at[pl.ds(q * CH, CH)]],
                              bufs[b], gsems[b]).wait()

    def wait_scatter(q, b):
        pltpu.make_async_copy(bufs[b], acc_s.at[col2_v.at[q]],
                              ssems[b]).wait()

    fire_gather(0, 0)
    fire_gather(1, 1)

    def slot(q, b, g):
        wait_gather(q, b)

        def scale(g2, _):
            nv = normb_v[pl.ds(q * CH + g2 * L, L)]
            for e in range(L):
                sv = jnp.full((L,), nv[e], jnp.float32)
                r = g2 * L + e
                for u in range(C // L):
                    bufs[b][r, pl.ds(u * L, L)] = (
                        bufs[b][r, pl.ds(u * L, L)] * sv)
            return 0
        lax.fori_loop(0, CH // L, scale, 0)

        pltpu.async_copy(bufs[b], acc_s.at[col2_v.at[q]], ssems[b], add=True)

        bg = (b + 2) % 3

        def refill():
            wait_scatter(q, bg)
            fire_gather(q + 2, bg)

        if b == 0:
            @pl.when(g > 0)
            def _():
                refill()

            @pl.when(g == 0)
            def _():
                fire_gather(q + 2, bg)
        elif b == 1:
            refill()
        else:
            @pl.when(g < BCH // 3 - 1)
            def _():
                refill()

    def outer_body(g, _):
        for b in range(3):
            slot(g * 3 + b, b, g)
        return 0
    lax.fori_loop(0, BCH // 3, outer_body, 0)

    slot_q = BCH - 1
    wait_gather(slot_q, 0)

    def scale_t(g2, _):
        nv = normb_v[pl.ds(slot_q * CH + g2 * L, L)]
        for e in range(L):
            sv = jnp.full((L,), nv[e], jnp.float32)
            r = g2 * L + e
            for u in range(C // L):
                bufs[0][r, pl.ds(u * L, L)] = bufs[0][r, pl.ds(u * L, L)] * sv
        return 0
    lax.fori_loop(0, CH // L, scale_t, 0)
    pltpu.async_copy(bufs[0], acc_s.at[col2_v.at[slot_q]], ssems[0], add=True)

    for b in range(3):
        wait_scatter(0, b)


def _agg_xh_kernel(rowf_h, row3_h, col3_h, ewf_h, xh_h, zeros_h,
                   a_out_h, norm_out_h,
                   dis_v, rowb_v, col2_v, normb_v,
                   b0, b1, b2,
                   acc_s, deg_s,
                   g0, g1, g2, s0, s1, s2, dsem):
    c = lax.axis_index("c")
    s = lax.axis_index("s")
    bufs = (b0, b1, b2)
    gsems = (g0, g1, g2)
    ssems = (s0, s1, s2)

    def zb(i, _):
        normb_v[pl.ds(i * L, L)] = jnp.zeros((L,), jnp.float32)
        return 0
    lax.fori_loop(0, 640 // L, zb, 0)

    @pl.when(s < 15)
    def _():
        pltpu.sync_copy(normb_v.at[pl.ds(0, 640)],
                        deg_s.at[pl.ds(s * 640, 640)])
        for i in range(5):
            pltpu.sync_copy(zeros_h, acc_s.at[pl.ds(s * 640 + i * 128, 128)])

    @pl.when(s == 15)
    def _():
        pltpu.sync_copy(normb_v.at[pl.ds(0, 400)],
                        deg_s.at[pl.ds(9600, 400)])
        for i in range(3):
            pltpu.sync_copy(zeros_h, acc_s.at[pl.ds(9600 + i * 128, 128)])
        pltpu.sync_copy(zeros_h.at[pl.ds(0, 16)],
                        acc_s.at[pl.ds(9984, 16)])
    plsc.subcore_barrier()

    def deg_blk(kb, _):
        k2 = s * (2 * NBLK // NW) + kb
        pltpu.sync_copy(row3_h.at[k2], col2_v)
        pltpu.sync_copy(ewf_h.at[k2], normb_v)

        def dfire(q, _):
            pltpu.async_copy(normb_v.at[pl.ds(q * CH, CH)],
                             deg_s.at[col2_v.at[q]], dsem, add=True)
            return 0
        lax.fori_loop(0, BCH, dfire, 0)

        def ddrain(q, _):
            pltpu.make_async_copy(normb_v.at[pl.ds(q * CH, CH)],
                                  deg_s.at[col2_v.at[q]], dsem).wait()
            return 0
        lax.fori_loop(0, BCH, ddrain, 0)
        return 0
    lax.fori_loop(0, 2 * NBLK // NW, deg_blk, 0)
    plsc.subcore_barrier()

    def newton(i, _):
        d = normb_v[pl.ds(i * L, L)]
        normb_v[pl.ds(i * L, L)] = _rsqrt_newton(d)
        return 0

    @pl.when(s < 15)
    def _():
        pltpu.sync_copy(deg_s.at[pl.ds(s * 640, 640)],
                        normb_v.at[pl.ds(0, 640)])
        lax.fori_loop(0, 640 // L, newton, 0)
        pltpu.sync_copy(normb_v.at[pl.ds(0, 640)],
                        deg_s.at[pl.ds(s * 640, 640)])

    @pl.when(s == 15)
    def _():
        pltpu.sync_copy(deg_s.at[pl.ds(9600, 400)],
                        normb_v.at[pl.ds(0, 400)])
        lax.fori_loop(0, 400 // L, newton, 0)
        pltpu.sync_copy(normb_v.at[pl.ds(0, 400)],
                        deg_s.at[pl.ds(9600, 400)])
    plsc.subcore_barrier()
    pltpu.sync_copy(deg_s, dis_v)

    off = jnp.full((L,), c * N, jnp.int32)

    def main_blk(kb, _):
        k2 = s * (2 * NBLK // NW) + kb
        pltpu.sync_copy(rowf_h.at[k2], rowb_v)
        pltpu.sync_copy(col3_h.at[k2], col2_v)
        pltpu.sync_copy(ewf_h.at[k2], normb_v)

        def normk(i, _):
            r16 = rowb_v[pl.ds(i * L, L)]
            c16 = col2_v[i // (CH // L), pl.ds((i % (CH // L)) * L, L)]
            w16 = normb_v[pl.ds(i * L, L)]
            a = plsc.load_gather(dis_v, [r16])
            b = plsc.load_gather(dis_v, [c16])
            normb_v[pl.ds(i * L, L)] = -(a * w16) * b
            rowb_v[pl.ds(i * L, L)] = r16 + off
            return 0
        lax.fori_loop(0, EPB // L, normk, 0)

        @pl.when(c == 0)
        def _():
            pltpu.sync_copy(normb_v, norm_out_h.at[k2])

        _agg_pipeline(xh_h, rowb_v, col2_v, normb_v, bufs, gsems, ssems,
                      acc_s)
        return 0
    lax.fori_loop(0, 2 * NBLK // NW, main_blk, 0)

    plsc.subcore_barrier()

    @pl.when(s < 15)
    def _():
        pltpu.sync_copy(acc_s.at[pl.ds(s * 640, 640)],
                        a_out_h.at[pl.ds(c * N + s * 640, 640)])

    @pl.when(s == 15)
    def _():
        pltpu.sync_copy(acc_s.at[pl.ds(9600, 400)],
                        a_out_h.at[pl.ds(c * N + 9600, 400)])


def _agg_xh(rowf, row3, col3, ewf, xh, zeros):
    f = pl.kernel(
        _agg_xh_kernel,
        out_type=(jax.ShapeDtypeStruct((2 * N, C), jnp.float32),
                  jax.ShapeDtypeStruct((NBLK, EPB), jnp.float32)),
        mesh=_mesh(),
        compiler_params=_params(),
        scratch_types=[
            pltpu.VMEM((N,), jnp.float32),
            pltpu.VMEM((EPB,), jnp.int32),
            pltpu.VMEM((BCH, CH), jnp.int32),
            pltpu.VMEM((EPB,), jnp.float32),
        ] + [pltpu.VMEM((CH, C), jnp.float32)] * 3
        + [pltpu.VMEM_SHARED((N, C), jnp.float32),
           pltpu.VMEM_SHARED((N,), jnp.float32)]
        + [pltpu.SemaphoreType.DMA] * 7,
    )
    return f(rowf, row3, col3, ewf, xh, zeros)


def _agg_rh_kernel(rowf_h, col3_h, normf_h, rh_h, zeros_h, a_out_h,
                   rowb_v, col2_v, normb_v,
                   b0, b1, b2,
                   acc_s,
                   g0, g1, g2, s0, s1, s2):
    c = lax.axis_index("c")
    s = lax.axis_index("s")
    w = c * NS + s
    bufs = (b0, b1, b2)
    gsems = (g0, g1, g2)
    ssems = (s0, s1, s2)

    @pl.when(s < 15)
    def _():
        for i in range(5):
            pltpu.sync_copy(zeros_h, acc_s.at[pl.ds(s * 640 + i * 128, 128)])

    @pl.when(s == 15)
    def _():
        for i in range(3):
            pltpu.sync_copy(zeros_h, acc_s.at[pl.ds(9600 + i * 128, 128)])
        pltpu.sync_copy(zeros_h.at[pl.ds(0, 16)],
                        acc_s.at[pl.ds(9984, 16)])
    plsc.subcore_barrier()

    def main_blk(kb, _):
        k2 = w * (NBLK // NW) + kb
        pltpu.sync_copy(rowf_h.at[k2], rowb_v)
        pltpu.sync_copy(col3_h.at[k2], col2_v)
        pltpu.sync_copy(normf_h.at[k2], normb_v)
        _agg_pipeline(rh_h, rowb_v, col2_v, normb_v, bufs, gsems, ssems,
                      acc_s)
        return 0
    lax.fori_loop(0, NBLK // NW, main_blk, 0)

    plsc.subcore_barrier()

    @pl.when(s < 15)
    def _():
        pltpu.sync_copy(acc_s.at[pl.ds(s * 640, 640)],
                        a_out_h.at[pl.ds(c * N + s * 640, 640)])

    @pl.when(s == 15)
    def _():
        pltpu.sync_copy(acc_s.at[pl.ds(9600, 400)],
                        a_out_h.at[pl.ds(c * N + 9600, 400)])


def _agg_rh(rowf, col3, normf, rh, zeros):
    f = pl.kernel(
        _agg_rh_kernel,
        out_type=jax.ShapeDtypeStruct((2 * N, C), jnp.float32),
        mesh=_mesh(),
        compiler_params=_params(),
        scratch_types=[
            pltpu.VMEM((EPB,), jnp.int32),
            pltpu.VMEM((BCH, CH), jnp.int32),
            pltpu.VMEM((EPB,), jnp.float32),
        ] + [pltpu.VMEM((CH, C), jnp.float32)] * 3
        + [pltpu.VMEM_SHARED((N, C), jnp.float32)]
        + [pltpu.SemaphoreType.DMA] * 6,
    )
    return f(rowf, col3, normf, rh, zeros)


def _gates_body(x_r, h_r, ax_r, ah_r,
                wz0a_r, wz0b_r, wz1a_r, wz1b_r, bz_r,
                wr0a_r, wr0b_r, wr1a_r, wr1b_r, br_r,
                wh0a_r, wh1a_r, bh_r,
                z_r, rh_r, cp_r):
    dot = functools.partial(jnp.dot, preferred_element_type=jnp.float32)
    xx, hh = x_r[...], h_r[...]
    ax, ah = ax_r[...], ah_r[...]
    gz = (dot(xx, wz0a_r[...]) + dot(hh, wz0b_r[...])
          + dot(ax, wz1a_r[...]) + dot(ah, wz1b_r[...]) + bz_r[...])
    gr = (dot(xx, wr0a_r[...]) + dot(hh, wr0b_r[...])
          + dot(ax, wr1a_r[...]) + dot(ah, wr1b_r[...]) + br_r[...])
    z_r[...] = jax.nn.sigmoid(gz)
    rh_r[...] = jax.nn.sigmoid(gr) * hh
    cp_r[...] = dot(xx, wh0a_r[...]) + dot(ax, wh1a_r[...]) + bh_r[...]


def _gates_tc(x, h, ax, ah, wz0a, wz0b, wz1a, wz1b, bz2,
              wr0a, wr0b, wr1a, wr1b, br2, wh0a, wh1a, bh2):
    nb = 10
    rs = pl.BlockSpec((N // nb, C), lambda i: (i, 0))
    ws = pl.BlockSpec((C, C), lambda i: (0, 0))
    bs = pl.BlockSpec((1, C), lambda i: (0, 0))
    return pl.pallas_call(
        _gates_body,
        grid=(nb,),
        in_specs=[rs, rs, rs, rs,
                  ws, ws, ws, ws, bs,
                  ws, ws, ws, ws, bs,
                  ws, ws, bs],
        out_specs=[rs, rs, rs],
        out_shape=[jax.ShapeDtypeStruct((N, C), jnp.float32)] * 3,
    )(x, h, ax, ah, wz0a, wz0b, wz1a, wz1b, bz2,
      wr0a, wr0b, wr1a, wr1b, br2, wh0a, wh1a, bh2)


def _out_body(z_r, rh_r, cp_r, h_r, ar0_r, ar1_r, wh0b_r, wh1b_r, o_r):
    dot = functools.partial(jnp.dot, preferred_element_type=jnp.float32)
    cand = jnp.tanh(cp_r[...] + dot(rh_r[...], wh0b_r[...])
                    + dot(ar0_r[...] + ar1_r[...], wh1b_r[...]))
    z = z_r[...]
    o_r[...] = (1.0 - z) * h_r[...] + z * cand


def _out_tc(z, rh, cp, h, ar0, ar1, wh0b, wh1b):
    nb = 10
    rs = pl.BlockSpec((N // nb, C), lambda i: (i, 0))
    ws = pl.BlockSpec((C, C), lambda i: (0, 0))
    return pl.pallas_call(
        _out_body,
        grid=(nb,),
        in_specs=[rs, rs, rs, rs, rs, rs, ws, ws],
        out_specs=rs,
        out_shape=jax.ShapeDtypeStruct((N, C), jnp.float32),
    )(z, rh, cp, h, ar0, ar1, wh0b, wh1b)


def kernel(x, edge_index, edge_weight, hidden_state,
           Wz0, Wz1, bz, Wr0, Wr1, br, Wh0, Wh1, bh):
    row = edge_index[0]
    col = edge_index[1]
    zeros = jnp.zeros((128, C), jnp.float32)
    xh = jnp.concatenate([x, hidden_state], axis=0)

    rowf = row.reshape(NBLK, EPB)
    row3 = row.reshape(NBLK, BCH, CH)
    col3 = col.reshape(NBLK, BCH, CH)
    ewf = edge_weight.reshape(NBLK, EPB)
    a_cat, normf = _agg_xh(rowf, row3, col3, ewf, xh, zeros)
    a_x, a_h = a_cat[:N], a_cat[N:]

    z, rh, cpre = _gates_tc(
        x, hidden_state, a_x, a_h,
        Wz0[:C], Wz0[C:], Wz1[:C], Wz1[C:], bz.reshape(1, C),
        Wr0[:C], Wr0[C:], Wr1[:C], Wr1[C:], br.reshape(1, C),
        Wh0[:C], Wh1[:C], bh.reshape(1, C))

    ar = _agg_rh(rowf, col3, normf, rh, zeros)

    return _out_tc(z, rh, cpre, hidden_state, ar[:N], ar[N:],
                   Wh0[C:], Wh1[C:])
